# baseline (speedup 1.0000x reference)
import functools

import jax
import jax.numpy as jnp
from jax import lax
from jax.experimental import pallas as pl
from jax.experimental.pallas import tpu as pltpu
from jax.experimental.pallas import tpu_sc as plsc

B = 16384
L = 50
D = 64
H = 256
V = 1000000

VP = 1 << 20
QUARTER = VP // 4
TW = 8192
TGRID = QUARTER // TW
LASTB = (V - 1) // TW

NC = 2
NS = 16
NW = NC * NS
ROWS_PER_W = B // NW
CB = 8
GP = 2
NG = CB // GP
NCHUNK = ROWS_PER_W // CB
IDXW = GP * L
NROW = NCHUNK * NG

PACKED_D = 32


def _pack_body(a_ref, b_ref, c_ref, d_ref, o_ref):
    parts = []
    for ref in (a_ref, b_ref, c_ref, d_ref):
        bits = jax.lax.bitcast_convert_type(ref[...], jnp.int32)
        lo = jax.lax.shift_right_logical(bits[:PACKED_D, :], 16)
        hi = bits[PACKED_D:, :] & jnp.int32(-65536)
        parts.append(lo | hi)
    o_ref[...] = jax.lax.bitcast_convert_type(
        jnp.concatenate(parts, axis=0).T, jnp.float32)


def _pack_table(emb_t):
    return pl.pallas_call(
        _pack_body,
        grid=(TGRID,),
        in_specs=[
            pl.BlockSpec((D, TW), lambda i: (0, i)),
            pl.BlockSpec((D, TW), lambda i: (0, i + TGRID)),
            pl.BlockSpec((D, TW), lambda i: (0, i + 2 * TGRID)),
            pl.BlockSpec((D, TW), lambda i: (0, jnp.minimum(i + 3 * TGRID, LASTB))),
        ],
        out_specs=pl.BlockSpec((TW, 4 * PACKED_D), lambda i: (i, 0)),
        out_shape=jax.ShapeDtypeStruct((QUARTER, 4 * PACKED_D), jnp.float32),
    )(emb_t, emb_t, emb_t, emb_t)


def _pool_body(x_hbm, sig_hbm, table_hbm, out_hbm,
               idx_v, sig_v, pooled_v, rows_refs, sem_g):
    wid = lax.axis_index("s") * NC + lax.axis_index("c")
    base = wid * ROWS_PER_W
    mask_hi = jnp.int32(-65536)

    def start_gathers(slot, c):
        cc = lax.rem(c, NCHUNK)
        for g in range(NG):
            pltpu.async_copy(table_hbm.at[idx_v.at[cc * NG + g]],
                             rows_refs[slot][g], sem_g)

    def wait_gathers(slot):
        for g in range(NG):
            pltpu.make_async_copy(table_hbm.at[idx_v.at[0]],
                                  rows_refs[slot][g], sem_g).wait()

    def compute(slot, c):
        sig_base = c * (CB * L)
        for b in range(CB):
            rows = rows_refs[slot][b // GP]
            r = b % GP

            def lbody(lb, acc):
                a0, a1, a2, a3 = acc
                for dl in range(10):
                    l = lb * 10 + dl
                    j = r * L + l
                    w = plsc.load_gather(
                        sig_v,
                        [jnp.full((16,), sig_base + b * L + l, jnp.int32)])
                    u0 = plsc.bitcast(rows[j, pl.ds(0, 16)], jnp.int32)
                    u1 = plsc.bitcast(rows[j, pl.ds(16, 16)], jnp.int32)
                    a0 = a0 + plsc.bitcast(u0 << 16, jnp.float32) * w
                    a1 = a1 + plsc.bitcast(u1 << 16, jnp.float32) * w
                    a2 = a2 + plsc.bitcast(u0 & mask_hi, jnp.float32) * w
                    a3 = a3 + plsc.bitcast(u1 & mask_hi, jnp.float32) * w
                return a0, a1, a2, a3

            z = jnp.zeros((16,), jnp.float32)
            a0, a1, a2, a3 = lax.fori_loop(0, L // 10, lbody, (z, z, z, z))
            pooled_v[b, pl.ds(0, 16)] = a0
            pooled_v[b, pl.ds(16, 16)] = a1
            pooled_v[b, pl.ds(32, 16)] = a2
            pooled_v[b, pl.ds(48, 16)] = a3
        pltpu.sync_copy(pooled_v, out_hbm.at[pl.ds(base + c * CB, CB)])

    pltpu.sync_copy(x_hbm.at[wid], idx_v)
    pltpu.sync_copy(sig_hbm.at[pl.ds(wid * (ROWS_PER_W * L), ROWS_PER_W * L)],
                    sig_v)

    start_gathers(0, 0)

    def chunk_pair(c2, carry):
        for s in range(2):
            c = c2 * 2 + s
            wait_gathers(s)
            start_gathers(s ^ 1, c + 1)
            compute(s, c)
        return carry

    lax.fori_loop(0, NCHUNK // 2, chunk_pair, 0)

    wait_gathers(NCHUNK % 2)


@functools.partial(
    pl.kernel,
    out_type=jax.ShapeDtypeStruct((B, D), jnp.float32),
    mesh=plsc.VectorSubcoreMesh(core_axis_name="c", subcore_axis_name="s"),
    compiler_params=pltpu.CompilerParams(needs_layout_passes=False,
                                         use_tc_tiling_on_sc=False),
    scratch_types=[
        pltpu.VMEM((NROW, IDXW), jnp.int32),
        pltpu.VMEM((ROWS_PER_W * L,), jnp.float32),
        pltpu.VMEM((CB, D), jnp.float32),
        [[pltpu.VMEM((IDXW, PACKED_D), jnp.float32) for _ in range(NG)]
         for _ in range(2)],
        pltpu.SemaphoreType.DMA,
    ],
)
def _pool_kernel(x_hbm, sig_hbm, table_hbm, out_hbm,
                 idx_v, sig_v, pooled_v, rows_refs, sem_g):
    _pool_body(x_hbm, sig_hbm, table_hbm, out_hbm,
               idx_v, sig_v, pooled_v, rows_refs, sem_g)


def _mlp_body(p_ref, w1_ref, b1_ref, w2_ref, b2_ref, o_ref):
    h = jnp.dot(p_ref[...], w1_ref[...], preferred_element_type=jnp.float32)
    h = jnp.maximum(h + b1_ref[...], 0.0)
    o = jnp.dot(h, w2_ref[...], preferred_element_type=jnp.float32)
    o_ref[...] = o + b2_ref[...]


def _mlp(pooled, W1, b1, W2, b2):
    BM = 2048
    return pl.pallas_call(
        _mlp_body,
        grid=(B // BM,),
        in_specs=[
            pl.BlockSpec((BM, D), lambda i: (i, 0)),
            pl.BlockSpec((D, H), lambda i: (0, 0)),
            pl.BlockSpec((1, H), lambda i: (0, 0)),
            pl.BlockSpec((H, 2), lambda i: (0, 0)),
            pl.BlockSpec((1, 2), lambda i: (0, 0)),
        ],
        out_specs=pl.BlockSpec((BM, 2), lambda i: (i, 0)),
        out_shape=jax.ShapeDtypeStruct((B, 2), jnp.float32),
    )(pooled, W1, b1, W2, b2)


def kernel(x, signal_strengths, embedding, W1, b1, W2, b2):
    emb_t = embedding.T
    packed = _pack_table(emb_t)
    tab = packed.reshape(VP, PACKED_D)
    xm = (x % QUARTER) * 4 + (x // QUARTER)
    x3 = xm.reshape(NW, NROW, IDXW)
    sig1 = signal_strengths.reshape(NW * ROWS_PER_W * L)
    pooled = _pool_kernel(x3, sig1, tab)
    return _mlp(pooled, W1, b1.reshape(1, H), W2, b2.reshape(1, 2))


# speedup vs baseline: 1.0030x; 1.0030x over previous
import functools

import jax
import jax.numpy as jnp
from jax import lax
from jax.experimental import pallas as pl
from jax.experimental.pallas import tpu as pltpu
from jax.experimental.pallas import tpu_sc as plsc

B = 16384
L = 50
D = 64
H = 256
V = 1000000

VP = 1 << 20
QUARTER = VP // 4
TW = 8192
TGRID = QUARTER // TW
LASTB = (V - 1) // TW

NC = 2
NS = 16
NW = NC * NS
ROWS_PER_W = B // NW
CB = 8
GP = 2
NG = CB // GP
NCHUNK = ROWS_PER_W // CB
IDXW = GP * L
NROW = NCHUNK * NG

PACKED_D = 32


def _pack_body(a_ref, b_ref, c_ref, d_ref, o_ref):
    parts = []
    for ref in (a_ref, b_ref, c_ref, d_ref):
        bits = jax.lax.bitcast_convert_type(ref[...], jnp.int32)
        lo = jax.lax.shift_right_logical(bits[:PACKED_D, :], 16)
        hi = bits[PACKED_D:, :] & jnp.int32(-65536)
        parts.append(lo | hi)
    o_ref[...] = jax.lax.bitcast_convert_type(
        jnp.concatenate(parts, axis=0).T, jnp.float32)


def _pack_table(emb_t):
    return pl.pallas_call(
        _pack_body,
        grid=(TGRID,),
        in_specs=[
            pl.BlockSpec((D, TW), lambda i: (0, i)),
            pl.BlockSpec((D, TW), lambda i: (0, i + TGRID)),
            pl.BlockSpec((D, TW), lambda i: (0, i + 2 * TGRID)),
            pl.BlockSpec((D, TW), lambda i: (0, jnp.minimum(i + 3 * TGRID, LASTB))),
        ],
        out_specs=pl.BlockSpec((TW, 4 * PACKED_D), lambda i: (i, 0)),
        out_shape=jax.ShapeDtypeStruct((QUARTER, 4 * PACKED_D), jnp.float32),
    )(emb_t, emb_t, emb_t, emb_t)


def _pool_body(x_hbm, sig_hbm, table_hbm, out_hbm,
               idx_v, sig_v, pooled_v, rows_refs, sem_g):
    wid = lax.axis_index("s") * NC + lax.axis_index("c")
    base = wid * ROWS_PER_W
    mask_hi = jnp.int32(-65536)

    def start_gathers(slot, c):
        cc = lax.rem(c, NCHUNK)
        for g in range(NG):
            pltpu.async_copy(table_hbm.at[idx_v.at[cc * NG + g]],
                             rows_refs[slot][g], sem_g)

    def wait_gathers(slot):
        for g in range(NG):
            pltpu.make_async_copy(table_hbm.at[idx_v.at[0]],
                                  rows_refs[slot][g], sem_g).wait()

    def compute(slot, c):
        sig_base = c * (CB * L)
        for b in range(CB):
            rows = rows_refs[slot][b // GP]
            r = b % GP

            def lbody(lb, acc):
                a0, a1, a2, a3 = acc
                for dl in range(10):
                    l = lb * 10 + dl
                    j = r * L + l
                    w = plsc.load_gather(
                        sig_v,
                        [jnp.full((16,), sig_base + b * L + l, jnp.int32)])
                    u0 = plsc.bitcast(rows[j, pl.ds(0, 16)], jnp.int32)
                    u1 = plsc.bitcast(rows[j, pl.ds(16, 16)], jnp.int32)
                    a0 = a0 + plsc.bitcast(u0 << 16, jnp.float32) * w
                    a1 = a1 + plsc.bitcast(u1 << 16, jnp.float32) * w
                    a2 = a2 + plsc.bitcast(u0 & mask_hi, jnp.float32) * w
                    a3 = a3 + plsc.bitcast(u1 & mask_hi, jnp.float32) * w
                return a0, a1, a2, a3

            z = jnp.zeros((16,), jnp.float32)
            a0, a1, a2, a3 = lax.fori_loop(0, L // 10, lbody, (z, z, z, z))
            pooled_v[b, pl.ds(0, 16)] = a0
            pooled_v[b, pl.ds(16, 16)] = a1
            pooled_v[b, pl.ds(32, 16)] = a2
            pooled_v[b, pl.ds(48, 16)] = a3
        pltpu.sync_copy(pooled_v, out_hbm.at[pl.ds(base + c * CB, CB)])

    pltpu.sync_copy(x_hbm.at[wid], idx_v)
    pltpu.sync_copy(sig_hbm.at[pl.ds(wid * (ROWS_PER_W * L), ROWS_PER_W * L)],
                    sig_v)

    start_gathers(0, 0)

    def chunk_pair(c2, carry):
        for s in range(2):
            c = c2 * 2 + s
            wait_gathers(s)
            start_gathers(s ^ 1, c + 1)
            compute(s, c)
        return carry

    lax.fori_loop(0, NCHUNK // 2, chunk_pair, 0)

    wait_gathers(NCHUNK % 2)


@functools.partial(
    pl.kernel,
    out_type=jax.ShapeDtypeStruct((B, D), jnp.float32),
    mesh=plsc.VectorSubcoreMesh(core_axis_name="c", subcore_axis_name="s"),
    compiler_params=pltpu.CompilerParams(needs_layout_passes=False,
                                         use_tc_tiling_on_sc=False),
    scratch_types=[
        pltpu.VMEM((NROW, IDXW), jnp.int32),
        pltpu.VMEM((ROWS_PER_W * L,), jnp.float32),
        pltpu.VMEM((CB, D), jnp.float32),
        [[pltpu.VMEM((IDXW, PACKED_D), jnp.float32) for _ in range(NG)]
         for _ in range(2)],
        pltpu.SemaphoreType.DMA,
    ],
)
def _pool_kernel(x_hbm, sig_hbm, table_hbm, out_hbm,
                 idx_v, sig_v, pooled_v, rows_refs, sem_g):
    _pool_body(x_hbm, sig_hbm, table_hbm, out_hbm,
               idx_v, sig_v, pooled_v, rows_refs, sem_g)


def _mlp_body(p_ref, w1_ref, b1_ref, w2_ref, b2_ref, o_ref):
    h = jnp.dot(p_ref[...], w1_ref[...], preferred_element_type=jnp.float32)
    h = jnp.maximum(h + b1_ref[...], 0.0)
    o = jnp.dot(h, w2_ref[...], preferred_element_type=jnp.float32)
    o_ref[...] = o + b2_ref[...]


def _mlp(pooled, W1, b1, W2p, b2p):
    BM = 2048
    return pl.pallas_call(
        _mlp_body,
        grid=(B // BM,),
        in_specs=[
            pl.BlockSpec((BM, D), lambda i: (i, 0)),
            pl.BlockSpec((D, H), lambda i: (0, 0)),
            pl.BlockSpec((1, H), lambda i: (0, 0)),
            pl.BlockSpec((H, 128), lambda i: (0, 0)),
            pl.BlockSpec((1, 128), lambda i: (0, 0)),
        ],
        out_specs=pl.BlockSpec((BM, 128), lambda i: (i, 0)),
        out_shape=jax.ShapeDtypeStruct((B, 128), jnp.float32),
    )(pooled, W1, b1, W2p, b2p)


def kernel(x, signal_strengths, embedding, W1, b1, W2, b2):
    emb_t = embedding.T
    packed = _pack_table(emb_t)
    tab = packed.reshape(VP, PACKED_D)
    xm = (x % QUARTER) * 4 + (x // QUARTER)
    x3 = xm.reshape(NW, NROW, IDXW)
    sig1 = signal_strengths.reshape(NW * ROWS_PER_W * L)
    pooled = _pool_kernel(x3, sig1, tab)
    W2p = jnp.zeros((H, 128), jnp.float32).at[:, :2].set(W2)
    b2p = jnp.zeros((1, 128), jnp.float32).at[0, :2].set(b2)
    out = _mlp(pooled, W1, b1.reshape(1, H), W2p, b2p)
    return out[:, :2]


# speedup vs baseline: 1.0986x; 1.0953x over previous
import functools

import jax
import jax.numpy as jnp
from jax import lax
from jax.experimental import pallas as pl
from jax.experimental.pallas import tpu as pltpu
from jax.experimental.pallas import tpu_sc as plsc

B = 16384
L = 50
D = 64
H = 256
V = 1000000

VP = 1 << 20
QUARTER = VP // 4
TW = 8192
TGRID = QUARTER // TW
LASTB = (V - 1) // TW

NC = 2
NS = 16
NW = NC * NS
ROWS_PER_W = B // NW
CB = 8
GP = 2
NG = CB // GP
NCHUNK = ROWS_PER_W // CB
IDXW = GP * L
NROW = NCHUNK * NG

PACKED_D = 32


def _pack_body(a_ref, b_ref, c_ref, d_ref, o_ref):
    parts = []
    for ref in (a_ref, b_ref, c_ref, d_ref):
        bits = jax.lax.bitcast_convert_type(ref[...], jnp.int32)
        lo = jax.lax.shift_right_logical(bits[:PACKED_D, :], 16)
        hi = bits[PACKED_D:, :] & jnp.int32(-65536)
        parts.append(lo | hi)
    o_ref[...] = jax.lax.bitcast_convert_type(
        jnp.concatenate(parts, axis=0).T, jnp.float32)


def _pack_table(emb_t):
    return pl.pallas_call(
        _pack_body,
        grid=(TGRID,),
        in_specs=[
            pl.BlockSpec((D, TW), lambda i: (0, i)),
            pl.BlockSpec((D, TW), lambda i: (0, i + TGRID)),
            pl.BlockSpec((D, TW), lambda i: (0, i + 2 * TGRID)),
            pl.BlockSpec((D, TW), lambda i: (0, jnp.minimum(i + 3 * TGRID, LASTB))),
        ],
        out_specs=pl.BlockSpec((TW, 4 * PACKED_D), lambda i: (i, 0)),
        out_shape=jax.ShapeDtypeStruct((QUARTER, 4 * PACKED_D), jnp.float32),
    )(emb_t, emb_t, emb_t, emb_t)


PBLK = 2048


def _prep_body(xt_ref, st_ref, xo_ref, so_ref):
    xt = xt_ref[...]
    xm = (xt % QUARTER) * 4 + xt // QUARTER
    xo_ref[...] = xm.T
    so_ref[...] = st_ref[...].T


def _prep(xt, sigt):
    return pl.pallas_call(
        _prep_body,
        grid=(B // PBLK,),
        in_specs=[
            pl.BlockSpec((L, PBLK), lambda i: (0, i)),
            pl.BlockSpec((L, PBLK), lambda i: (0, i)),
        ],
        out_specs=[
            pl.BlockSpec((PBLK, L), lambda i: (i, 0)),
            pl.BlockSpec((PBLK, L), lambda i: (i, 0)),
        ],
        out_shape=[
            jax.ShapeDtypeStruct((B, L), jnp.int32),
            jax.ShapeDtypeStruct((B, L), jnp.float32),
        ],
    )(xt, sigt)


def _pool_body(x_hbm, sig_hbm, table_hbm, out_hbm,
               idx_v, sig_v, pooled_v, rows_refs, sem_g):
    wid = lax.axis_index("s") * NC + lax.axis_index("c")
    base = wid * ROWS_PER_W
    mask_hi = jnp.int32(-65536)

    def start_gathers(slot, c):
        cc = lax.rem(c, NCHUNK)
        for g in range(NG):
            pltpu.async_copy(table_hbm.at[idx_v.at[cc * NG + g]],
                             rows_refs[slot][g], sem_g)

    def wait_gathers(slot):
        for g in range(NG):
            pltpu.make_async_copy(table_hbm.at[idx_v.at[0]],
                                  rows_refs[slot][g], sem_g).wait()

    def compute(slot, c):
        sig_base = c * (CB * L)
        for b in range(CB):
            rows = rows_refs[slot][b // GP]
            r = b % GP

            def lbody(lb, acc):
                a0, a1, a2, a3 = acc
                for dl in range(5):
                    l = lb * 5 + dl
                    j = r * L + l
                    w = plsc.load_gather(
                        sig_v,
                        [jnp.full((16,), sig_base + b * L + l, jnp.int32)])
                    u0 = plsc.bitcast(rows[j, pl.ds(0, 16)], jnp.int32)
                    u1 = plsc.bitcast(rows[j, pl.ds(16, 16)], jnp.int32)
                    a0 = a0 + plsc.bitcast(u0 << 16, jnp.float32) * w
                    a1 = a1 + plsc.bitcast(u1 << 16, jnp.float32) * w
                    a2 = a2 + plsc.bitcast(u0 & mask_hi, jnp.float32) * w
                    a3 = a3 + plsc.bitcast(u1 & mask_hi, jnp.float32) * w
                return a0, a1, a2, a3

            z = jnp.zeros((16,), jnp.float32)
            a0, a1, a2, a3 = lax.fori_loop(0, L // 5, lbody, (z, z, z, z))
            pooled_v[b, pl.ds(0, 16)] = a0
            pooled_v[b, pl.ds(16, 16)] = a1
            pooled_v[b, pl.ds(32, 16)] = a2
            pooled_v[b, pl.ds(48, 16)] = a3
        pltpu.sync_copy(pooled_v, out_hbm.at[pl.ds(base + c * CB, CB)])

    pltpu.sync_copy(x_hbm.at[wid], idx_v)
    pltpu.sync_copy(sig_hbm.at[pl.ds(wid * (ROWS_PER_W * L), ROWS_PER_W * L)],
                    sig_v)

    start_gathers(0, 0)

    def chunk_pair(c2, carry):
        for s in range(2):
            c = c2 * 2 + s
            wait_gathers(s)
            start_gathers(s ^ 1, c + 1)
            compute(s, c)
        return carry

    lax.fori_loop(0, NCHUNK // 2, chunk_pair, 0)

    wait_gathers(NCHUNK % 2)


@functools.partial(
    pl.kernel,
    out_type=jax.ShapeDtypeStruct((B, D), jnp.float32),
    mesh=plsc.VectorSubcoreMesh(core_axis_name="c", subcore_axis_name="s"),
    compiler_params=pltpu.CompilerParams(needs_layout_passes=False,
                                         use_tc_tiling_on_sc=False),
    scratch_types=[
        pltpu.VMEM((NROW, IDXW), jnp.int32),
        pltpu.VMEM((ROWS_PER_W * L,), jnp.float32),
        pltpu.VMEM((CB, D), jnp.float32),
        [[pltpu.VMEM((IDXW, PACKED_D), jnp.float32) for _ in range(NG)]
         for _ in range(2)],
        pltpu.SemaphoreType.DMA,
    ],
)
def _pool_kernel(x_hbm, sig_hbm, table_hbm, out_hbm,
                 idx_v, sig_v, pooled_v, rows_refs, sem_g):
    _pool_body(x_hbm, sig_hbm, table_hbm, out_hbm,
               idx_v, sig_v, pooled_v, rows_refs, sem_g)


def _mlp_body(p_ref, w1_ref, b1_ref, w2_ref, b2_ref, o_ref):
    h = jnp.dot(p_ref[...], w1_ref[...], preferred_element_type=jnp.float32)
    h = jnp.maximum(h + b1_ref[...], 0.0)
    o = jnp.dot(h, w2_ref[...], preferred_element_type=jnp.float32)
    o_ref[...] = o + b2_ref[...]


def _mlp(pooled, W1, b1, W2p, b2p):
    BM = 2048
    return pl.pallas_call(
        _mlp_body,
        grid=(B // BM,),
        in_specs=[
            pl.BlockSpec((BM, D), lambda i: (i, 0)),
            pl.BlockSpec((D, H), lambda i: (0, 0)),
            pl.BlockSpec((1, H), lambda i: (0, 0)),
            pl.BlockSpec((H, 128), lambda i: (0, 0)),
            pl.BlockSpec((1, 128), lambda i: (0, 0)),
        ],
        out_specs=pl.BlockSpec((BM, 128), lambda i: (i, 0)),
        out_shape=jax.ShapeDtypeStruct((B, 128), jnp.float32),
    )(pooled, W1, b1, W2p, b2p)


def kernel(x, signal_strengths, embedding, W1, b1, W2, b2):
    emb_t = embedding.T
    packed = _pack_table(emb_t)
    tab = packed.reshape(VP, PACKED_D)
    xm, sigm = _prep(x.T, signal_strengths.T)
    x3 = xm.reshape(NW, NROW, IDXW)
    sig1 = sigm.reshape(NW * ROWS_PER_W * L)
    pooled = _pool_kernel(x3, sig1, tab)
    W2p = jnp.zeros((H, 128), jnp.float32).at[:, :2].set(W2)
    b2p = jnp.zeros((1, 128), jnp.float32).at[0, :2].set(b2)
    out = _mlp(pooled, W1, b1.reshape(1, H), W2p, b2p)
    return out[:, :2]


# speedup vs baseline: 1.0992x; 1.0006x over previous
import functools

import jax
import jax.numpy as jnp
from jax import lax
from jax.experimental import pallas as pl
from jax.experimental.pallas import tpu as pltpu
from jax.experimental.pallas import tpu_sc as plsc

B = 16384
L = 50
D = 64
H = 256
V = 1000000

VP = 1 << 20
QUARTER = VP // 4
TW = 16384
TGRID = QUARTER // TW
LASTB = (V - 1) // TW

NC = 2
NS = 16
NW = NC * NS
ROWS_PER_W = B // NW
CB = 8
GP = 2
NG = CB // GP
NCHUNK = ROWS_PER_W // CB
IDXW = GP * L
NROW = NCHUNK * NG

PACKED_D = 32


def _pack_body(a_ref, b_ref, c_ref, d_ref, o_ref):
    parts = []
    for ref in (a_ref, b_ref, c_ref, d_ref):
        bits = jax.lax.bitcast_convert_type(ref[...], jnp.int32)
        lo = jax.lax.shift_right_logical(bits[:PACKED_D, :], 16)
        hi = bits[PACKED_D:, :] & jnp.int32(-65536)
        parts.append(lo | hi)
    o_ref[...] = jax.lax.bitcast_convert_type(
        jnp.concatenate(parts, axis=0).T, jnp.float32)


def _pack_table(emb_t):
    return pl.pallas_call(
        _pack_body,
        grid=(TGRID,),
        in_specs=[
            pl.BlockSpec((D, TW), lambda i: (0, i)),
            pl.BlockSpec((D, TW), lambda i: (0, i + TGRID)),
            pl.BlockSpec((D, TW), lambda i: (0, i + 2 * TGRID)),
            pl.BlockSpec((D, TW), lambda i: (0, jnp.minimum(i + 3 * TGRID, LASTB))),
        ],
        out_specs=pl.BlockSpec((TW, 4 * PACKED_D), lambda i: (i, 0)),
        out_shape=jax.ShapeDtypeStruct((QUARTER, 4 * PACKED_D), jnp.float32),
    )(emb_t, emb_t, emb_t, emb_t)


PBLK = 2048


def _prep_body(xt_ref, st_ref, xo_ref, so_ref):
    xt = xt_ref[...]
    xm = (xt % QUARTER) * 4 + xt // QUARTER
    xo_ref[...] = xm.T
    so_ref[...] = st_ref[...].T


def _prep(xt, sigt):
    return pl.pallas_call(
        _prep_body,
        grid=(B // PBLK,),
        in_specs=[
            pl.BlockSpec((L, PBLK), lambda i: (0, i)),
            pl.BlockSpec((L, PBLK), lambda i: (0, i)),
        ],
        out_specs=[
            pl.BlockSpec((PBLK, L), lambda i: (i, 0)),
            pl.BlockSpec((PBLK, L), lambda i: (i, 0)),
        ],
        out_shape=[
            jax.ShapeDtypeStruct((B, L), jnp.int32),
            jax.ShapeDtypeStruct((B, L), jnp.float32),
        ],
    )(xt, sigt)


def _pool_body(x_hbm, sig_hbm, table_hbm, out_hbm,
               idx_v, sig_v, pooled_v, rows_refs, sem_g):
    wid = lax.axis_index("s") * NC + lax.axis_index("c")
    base = wid * ROWS_PER_W
    mask_hi = jnp.int32(-65536)

    def start_gathers(slot, c):
        cc = lax.rem(c, NCHUNK)
        for g in range(NG):
            pltpu.async_copy(table_hbm.at[idx_v.at[cc * NG + g]],
                             rows_refs[slot][g], sem_g)

    def wait_gathers(slot):
        for g in range(NG):
            pltpu.make_async_copy(table_hbm.at[idx_v.at[0]],
                                  rows_refs[slot][g], sem_g).wait()

    def compute(slot, c):
        sig_base = c * (CB * L)
        for b in range(CB):
            rows = rows_refs[slot][b // GP]
            r = b % GP

            def lbody(lb, acc):
                a0, a1, a2, a3 = acc
                for dl in range(5):
                    l = lb * 5 + dl
                    j = r * L + l
                    w = plsc.load_gather(
                        sig_v,
                        [jnp.full((16,), sig_base + b * L + l, jnp.int32)])
                    u0 = plsc.bitcast(rows[j, pl.ds(0, 16)], jnp.int32)
                    u1 = plsc.bitcast(rows[j, pl.ds(16, 16)], jnp.int32)
                    a0 = a0 + plsc.bitcast(u0 << 16, jnp.float32) * w
                    a1 = a1 + plsc.bitcast(u1 << 16, jnp.float32) * w
                    a2 = a2 + plsc.bitcast(u0 & mask_hi, jnp.float32) * w
                    a3 = a3 + plsc.bitcast(u1 & mask_hi, jnp.float32) * w
                return a0, a1, a2, a3

            z = jnp.zeros((16,), jnp.float32)
            a0, a1, a2, a3 = lax.fori_loop(0, L // 5, lbody, (z, z, z, z))
            pooled_v[b, pl.ds(0, 16)] = a0
            pooled_v[b, pl.ds(16, 16)] = a1
            pooled_v[b, pl.ds(32, 16)] = a2
            pooled_v[b, pl.ds(48, 16)] = a3
        pltpu.sync_copy(pooled_v, out_hbm.at[pl.ds(base + c * CB, CB)])

    pltpu.sync_copy(x_hbm.at[wid], idx_v)
    pltpu.sync_copy(sig_hbm.at[pl.ds(wid * (ROWS_PER_W * L), ROWS_PER_W * L)],
                    sig_v)

    start_gathers(0, 0)

    def chunk_pair(c2, carry):
        for s in range(2):
            c = c2 * 2 + s
            wait_gathers(s)
            start_gathers(s ^ 1, c + 1)
            compute(s, c)
        return carry

    lax.fori_loop(0, NCHUNK // 2, chunk_pair, 0)

    wait_gathers(NCHUNK % 2)


@functools.partial(
    pl.kernel,
    out_type=jax.ShapeDtypeStruct((B, D), jnp.float32),
    mesh=plsc.VectorSubcoreMesh(core_axis_name="c", subcore_axis_name="s"),
    compiler_params=pltpu.CompilerParams(needs_layout_passes=False,
                                         use_tc_tiling_on_sc=False),
    scratch_types=[
        pltpu.VMEM((NROW, IDXW), jnp.int32),
        pltpu.VMEM((ROWS_PER_W * L,), jnp.float32),
        pltpu.VMEM((CB, D), jnp.float32),
        [[pltpu.VMEM((IDXW, PACKED_D), jnp.float32) for _ in range(NG)]
         for _ in range(2)],
        pltpu.SemaphoreType.DMA,
    ],
)
def _pool_kernel(x_hbm, sig_hbm, table_hbm, out_hbm,
                 idx_v, sig_v, pooled_v, rows_refs, sem_g):
    _pool_body(x_hbm, sig_hbm, table_hbm, out_hbm,
               idx_v, sig_v, pooled_v, rows_refs, sem_g)


def _mlp_body(p_ref, w1_ref, b1_ref, w2_ref, b2_ref, o_ref):
    h = jnp.dot(p_ref[...], w1_ref[...], preferred_element_type=jnp.float32)
    h = jnp.maximum(h + b1_ref[...], 0.0)
    o = jnp.dot(h, w2_ref[...], preferred_element_type=jnp.float32)
    o_ref[...] = o + b2_ref[...]


def _mlp(pooled, W1, b1, W2p, b2p):
    BM = 2048
    return pl.pallas_call(
        _mlp_body,
        grid=(B // BM,),
        in_specs=[
            pl.BlockSpec((BM, D), lambda i: (i, 0)),
            pl.BlockSpec((D, H), lambda i: (0, 0)),
            pl.BlockSpec((1, H), lambda i: (0, 0)),
            pl.BlockSpec((H, 128), lambda i: (0, 0)),
            pl.BlockSpec((1, 128), lambda i: (0, 0)),
        ],
        out_specs=pl.BlockSpec((BM, 128), lambda i: (i, 0)),
        out_shape=jax.ShapeDtypeStruct((B, 128), jnp.float32),
    )(pooled, W1, b1, W2p, b2p)


def kernel(x, signal_strengths, embedding, W1, b1, W2, b2):
    emb_t = embedding.T
    packed = _pack_table(emb_t)
    tab = packed.reshape(VP, PACKED_D)
    xm, sigm = _prep(x.T, signal_strengths.T)
    x3 = xm.reshape(NW, NROW, IDXW)
    sig1 = sigm.reshape(NW * ROWS_PER_W * L)
    pooled = _pool_kernel(x3, sig1, tab)
    W2p = jnp.zeros((H, 128), jnp.float32).at[:, :2].set(W2)
    b2p = jnp.zeros((1, 128), jnp.float32).at[0, :2].set(b2)
    out = _mlp(pooled, W1, b1.reshape(1, H), W2p, b2p)
    return out[:, :2]
